# baseline (speedup 1.0000x reference)
import functools

import jax
import jax.numpy as jnp
from jax import lax
from jax.experimental import pallas as pl
from jax.experimental.pallas import tpu as pltpu
from jax.experimental.pallas import tpu_sc as plsc

NC = 2
NS = 16
NW = NC * NS
L = 16

B = 16384
D = 16
BPW = B // NW
IDXW = 128
NCHUNK = BPW // IDXW


def _sc_body(uids_hbm, iids_hbm, utab_hbm, itab_hbm, out_hbm,
             uidx_v, iidx_v, urows_v, irows_v, out_v, sem):
    wid = lax.axis_index("s") * NC + lax.axis_index("c")

    pltpu.sync_copy(uids_hbm.at[wid], uidx_v)
    pltpu.sync_copy(iids_hbm.at[wid], iidx_v)

    copies = []
    for j in range(NCHUNK):
        copies.append(pltpu.async_copy(
            utab_hbm.at[uidx_v.at[j]],
            urows_v.at[pl.ds(j * IDXW, IDXW)], sem))
        copies.append(pltpu.async_copy(
            itab_hbm.at[iidx_v.at[j]],
            irows_v.at[pl.ds(j * IDXW, IDXW)], sem))
    for c in copies:
        c.wait()

    lane = lax.iota(jnp.int32, L)

    def block(k, carry):
        row0 = k * L
        rows = lane + row0
        acc = jnp.zeros((L,), jnp.float32)
        for d in range(D):
            cols = jnp.full((L,), d, jnp.int32)
            u = plsc.load_gather(urows_v, [rows, cols])
            it = plsc.load_gather(irows_v, [rows, cols])
            acc = acc + u * it
        out_v[pl.ds(row0, L)] = acc
        return carry

    lax.fori_loop(0, BPW // L, block, 0)

    pltpu.sync_copy(out_v, out_hbm.at[wid])


def kernel(user_ids, item_ids, user_table, item_table):
    mesh = plsc.VectorSubcoreMesh(core_axis_name="c", subcore_axis_name="s")

    sc_call = functools.partial(
        pl.kernel,
        out_type=jax.ShapeDtypeStruct((NW, BPW), jnp.float32),
        mesh=mesh,
        scratch_types=[
            pltpu.VMEM((NCHUNK, IDXW), jnp.int32),
            pltpu.VMEM((NCHUNK, IDXW), jnp.int32),
            pltpu.VMEM((BPW, D), jnp.float32),
            pltpu.VMEM((BPW, D), jnp.float32),
            pltpu.VMEM((BPW,), jnp.float32),
            pltpu.SemaphoreType.DMA,
        ],
        compiler_params=pltpu.CompilerParams(
            needs_layout_passes=False, use_tc_tiling_on_sc=False),
    )(_sc_body)

    uids = user_ids.astype(jnp.int32).reshape(NW, NCHUNK, IDXW)
    iids = item_ids.astype(jnp.int32).reshape(NW, NCHUNK, IDXW)
    out = sc_call(uids, iids, user_table, item_table)
    return out.reshape(B)
